# Initial kernel scaffold; baseline (speedup 1.0000x reference)
import jax
import jax.numpy as jnp
from jax.experimental import pallas as pl


def kernel(tokens, table, W, b):
    raise NotImplementedError("write your pallas kernel here")


# speedup vs baseline: 15.5424x; 15.5424x over previous
import functools

import jax
import jax.numpy as jnp
from jax import lax
from jax.experimental import pallas as pl
from jax.experimental.pallas import tpu as pltpu
from jax.experimental.pallas import tpu_sc as plsc

L = 16
NC, NS = 2, 16
NW = NC * NS
VP = 2 * L


def _make_kernel(B, SEQ, VOCAB, EMB):
    rows = B // NW
    chunks = rows // L

    mesh = plsc.VectorSubcoreMesh(core_axis_name="c", subcore_axis_name="s")

    @functools.partial(
        pl.kernel,
        out_type=jax.ShapeDtypeStruct((B,), jnp.float32),
        mesh=mesh,
        compiler_params=pltpu.CompilerParams(needs_layout_passes=False),
        scratch_types=[
            pltpu.VMEM((SEQ * rows,), jnp.int32),
            pltpu.VMEM((EMB * VP,), jnp.float32),
            pltpu.VMEM((EMB * L,), jnp.float32),
            pltpu.VMEM((L,), jnp.float32),
            pltpu.VMEM((VP,), jnp.float32),
            pltpu.VMEM((rows,), jnp.float32),
            pltpu.SemaphoreType.DMA,
        ],
    )
    def sc_kernel(tok_hbm, tt_hbm, wb_hbm, b_hbm, out_hbm,
                  tok_v, tt_v, wb_v, b_v, lut_v, out_v, sem):
        wid = lax.axis_index("s") * NC + lax.axis_index("c")

        tok_cp = pltpu.async_copy(tok_hbm.at[wid], tok_v, sem)
        pltpu.sync_copy(tt_hbm, tt_v)
        pltpu.sync_copy(wb_hbm, wb_v)
        pltpu.sync_copy(b_hbm, b_v)

        a0 = jnp.zeros((L,), jnp.float32)
        a1 = jnp.zeros((L,), jnp.float32)
        for d in range(EMB):
            w = wb_v[pl.ds(d * L, L)]
            a0 = a0 + tt_v[pl.ds(d * VP, L)] * w
            a1 = a1 + tt_v[pl.ds(d * VP + L, L)] * w
        bb = b_v[...]
        inv_seq = jnp.float32(1.0 / SEQ)
        lut_v[pl.ds(0, L)] = (a0 + bb) * inv_seq
        lut_v[pl.ds(L, L)] = (a1 + bb) * inv_seq

        tok_cp.wait()

        for j in range(chunks):
            g = plsc.load_gather(lut_v, [tok_v[pl.ds(j * L, L)]])
            for s in range(1, SEQ):
                idx = tok_v[pl.ds(s * rows + j * L, L)]
                g = g + plsc.load_gather(lut_v, [idx])
            out_v[pl.ds(j * L, L)] = 1.0 / (1.0 + jnp.exp(-g))

        pltpu.sync_copy(out_v, out_hbm.at[pl.ds(wid * rows, rows)])

    return sc_kernel


def kernel(tokens, table, W, b):
    B, SEQ = tokens.shape
    VOCAB, EMB = table.shape
    rows = B // NW

    tok_w = tokens.reshape(NW, rows, SEQ).transpose(0, 2, 1).reshape(NW, SEQ * rows)
    tt = jnp.pad(table.T, ((0, 0), (0, VP - VOCAB))).reshape(-1)
    wb = jnp.broadcast_to(W, (EMB, L)).reshape(-1)
    b16 = jnp.broadcast_to(b, (L,))

    out = _make_kernel(B, SEQ, VOCAB, EMB)(tok_w, tt, wb, b16)
    return out.reshape(B, 1)
